# baseline (speedup 1.0000x reference)
import functools

import jax
import jax.numpy as jnp
from jax import lax
from jax.experimental import pallas as pl
from jax.experimental.pallas import tpu as pltpu
from jax.experimental.pallas import tpu_sc as plsc

NC = 2
NS = 16
LANES = 16
EPS = 1e-12


def _rsqrt_fast(x):
    iv = lax.bitcast_convert_type(x, jnp.int32)
    seed = jnp.int32(0x5F3759DF) - lax.shift_right_logical(iv, 1)
    y = lax.bitcast_convert_type(seed, jnp.float32)
    half = x * jnp.float32(0.5)
    for _ in range(3):
        y = y * (jnp.float32(1.5) - half * y * y)
    return y


def _lane_sum(v):
    parts = [v[l] for l in range(LANES)]
    while len(parts) > 1:
        parts = [parts[i] + parts[i + 1] for i in range(0, len(parts), 2)]
    return parts[0]


def _make_sc_kernel(B, S, V, H, P):
    NW = NC * NS
    assert (S % NW) == 0 and P == S // NW and H % LANES == 0
    NG = H // LANES

    mesh = plsc.VectorSubcoreMesh(core_axis_name="c", subcore_axis_name="s")

    @functools.partial(
        pl.kernel,
        mesh=mesh,
        out_type=(
            jax.ShapeDtypeStruct((B * S, H), jnp.float32),
            jax.ShapeDtypeStruct((B * S, H), jnp.float32),
        ),
        scratch_types=[
            pltpu.VMEM((P, H), jnp.float32),
            pltpu.VMEM((P, H), jnp.float32),
            pltpu.VMEM((P,), jnp.int32),
            pltpu.VMEM((P + LANES,), jnp.float32),
            pltpu.VMEM((2, H), jnp.float32),
            pltpu.VMEM((H,), jnp.float32),
            pltpu.VMEM((H,), jnp.float32),
            pltpu.VMEM((H,), jnp.float32),
            pltpu.SemaphoreType.DMA,
        ],
    )
    def k(ids_hbm, tt_hbm, word_hbm, pos_hbm, type_hbm, gamma_hbm, beta_hbm,
          emb_out, word_out,
          pos_v, rows_v, idx_v, tt_v, type_v, td_v, g_v, b_v, sem):
        wid = lax.axis_index("s") * NC + lax.axis_index("c")
        p0 = wid * P

        pltpu.sync_copy(pos_hbm.at[pl.ds(p0, P)], pos_v)
        pltpu.sync_copy(type_hbm, type_v)
        pltpu.sync_copy(gamma_hbm, g_v)
        pltpu.sync_copy(beta_hbm, b_v)
        for j in range(NG):
            dsj = pl.ds(j * LANES, LANES)
            td_v[dsj] = type_v[1, dsj] - type_v[0, dsj]

        for b in range(B):
            base = b * S + p0
            pltpu.sync_copy(ids_hbm.at[pl.ds(base, P)], idx_v)
            pltpu.sync_copy(tt_hbm.at[pl.ds(base, P)], tt_v.at[pl.ds(0, P)])
            pltpu.async_copy(word_hbm.at[idx_v], rows_v, sem).wait()
            pltpu.sync_copy(rows_v, word_out.at[pl.ds(base, P)])

            def token_body(i, _):
                ttb = tt_v[pl.ds(i, LANES)][0]
                s = jnp.zeros((LANES,), jnp.float32)
                q = jnp.zeros((LANES,), jnp.float32)
                for j in range(NG):
                    dsj = pl.ds(j * LANES, LANES)
                    v = rows_v[i, dsj] + pos_v[i, dsj] + (
                        type_v[0, dsj] + ttb * td_v[dsj])
                    rows_v[i, dsj] = v
                    s = s + v
                    q = q + v * v
                inv_h = jnp.float32(1.0 / H)
                mean = _lane_sum(s) * inv_h
                var = _lane_sum(q) * inv_h - mean * mean
                rstd = _rsqrt_fast(var + jnp.float32(EPS))
                for j in range(NG):
                    dsj = pl.ds(j * LANES, LANES)
                    v = rows_v[i, dsj]
                    rows_v[i, dsj] = (v - mean) * rstd * g_v[dsj] + b_v[dsj]
                return 0

            lax.fori_loop(0, P, token_body, 0)
            pltpu.sync_copy(rows_v, emb_out.at[pl.ds(base, P)])

    return k


def kernel(input_ids, token_type_ids, word_emb, pos_emb, type_emb, gamma, beta):
    B, S = input_ids.shape
    V, H = word_emb.shape
    P = S // (NC * NS)
    ids_flat = input_ids.reshape(-1).astype(jnp.int32)
    tt_flat = token_type_ids.reshape(-1).astype(jnp.float32)
    k = _make_sc_kernel(B, S, V, H, P)
    emb, words = k(ids_flat, tt_flat, word_emb, pos_emb, type_emb, gamma, beta)
    return emb.reshape(B, S, H), words.reshape(B, S, H)


# speedup vs baseline: 1.2505x; 1.2505x over previous
import functools

import jax
import jax.numpy as jnp
from jax import lax
from jax.experimental import pallas as pl
from jax.experimental.pallas import tpu as pltpu
from jax.experimental.pallas import tpu_sc as plsc

NC = 2
NS = 16
LANES = 16
EPS = 1e-12


def _rsqrt_fast(x):
    iv = lax.bitcast_convert_type(x, jnp.int32)
    seed = jnp.int32(0x5F3759DF) - lax.shift_right_logical(iv, 1)
    y = lax.bitcast_convert_type(seed, jnp.float32)
    half = x * jnp.float32(0.5)
    for _ in range(3):
        y = y * (jnp.float32(1.5) - half * y * y)
    return y


def _lane_sum(v):
    parts = [v[l] for l in range(LANES)]
    while len(parts) > 1:
        parts = [parts[i] + parts[i + 1] for i in range(0, len(parts), 2)]
    return parts[0]


def _make_sc_kernel(B, S, V, H, P):
    NW = NC * NS
    assert (S % NW) == 0 and P == S // NW and H % LANES == 0
    NG = H // LANES

    mesh = plsc.VectorSubcoreMesh(core_axis_name="c", subcore_axis_name="s")

    @functools.partial(
        pl.kernel,
        mesh=mesh,
        out_type=(
            jax.ShapeDtypeStruct((B * S, H), jnp.float32),
            jax.ShapeDtypeStruct((B * S, H), jnp.float32),
        ),
        scratch_types=[
            pltpu.VMEM((P, H), jnp.float32),
            pltpu.VMEM((P, H), jnp.float32),
            pltpu.VMEM((P,), jnp.int32),
            pltpu.VMEM((P + LANES,), jnp.float32),
            pltpu.VMEM((2, H), jnp.float32),
            pltpu.VMEM((H,), jnp.float32),
            pltpu.VMEM((H,), jnp.float32),
            pltpu.VMEM((H,), jnp.float32),
            pltpu.SemaphoreType.DMA,
        ],
    )
    def k(ids_hbm, tt_hbm, word_hbm, pos_hbm, type_hbm, gamma_hbm, beta_hbm,
          emb_out, word_out,
          pos_v, rows_v, idx_v, tt_v, type_v, td_v, g_v, b_v, sem):
        wid = lax.axis_index("s") * NC + lax.axis_index("c")
        p0 = wid * P

        pltpu.sync_copy(pos_hbm.at[pl.ds(p0, P)], pos_v)
        pltpu.sync_copy(type_hbm, type_v)
        pltpu.sync_copy(gamma_hbm, g_v)
        pltpu.sync_copy(beta_hbm, b_v)
        for j in range(NG):
            dsj = pl.ds(j * LANES, LANES)
            td_v[dsj] = type_v[1, dsj] - type_v[0, dsj]

        @plsc.parallel_loop(0, P)
        def _fold(i):
            for j in range(NG):
                dsj = pl.ds(j * LANES, LANES)
                pos_v[i, dsj] = pos_v[i, dsj] + type_v[0, dsj]

        def batch_body(b, _):
            base = b * S + p0
            pltpu.sync_copy(ids_hbm.at[pl.ds(base, P)], idx_v)
            pltpu.sync_copy(tt_hbm.at[pl.ds(base, P)], tt_v.at[pl.ds(0, P)])
            pltpu.async_copy(word_hbm.at[idx_v], rows_v, sem).wait()
            pltpu.sync_copy(rows_v, word_out.at[pl.ds(base, P)])

            @plsc.parallel_loop(0, P)
            def _token(i):
                ttb = tt_v[pl.ds(i, LANES)][0]
                acc_s = [jnp.zeros((LANES,), jnp.float32) for _ in range(4)]
                acc_q = [jnp.zeros((LANES,), jnp.float32) for _ in range(4)]
                vs = []
                for j in range(NG):
                    dsj = pl.ds(j * LANES, LANES)
                    v = rows_v[i, dsj] + (pos_v[i, dsj] + ttb * td_v[dsj])
                    vs.append(v)
                    acc_s[j & 3] = acc_s[j & 3] + v
                    acc_q[j & 3] = acc_q[j & 3] + v * v
                s = (acc_s[0] + acc_s[1]) + (acc_s[2] + acc_s[3])
                q = (acc_q[0] + acc_q[1]) + (acc_q[2] + acc_q[3])
                inv_h = jnp.float32(1.0 / H)
                mean = _lane_sum(s) * inv_h
                var = _lane_sum(q) * inv_h - mean * mean
                rstd = _rsqrt_fast(var + jnp.float32(EPS))
                for j in range(NG):
                    dsj = pl.ds(j * LANES, LANES)
                    rows_v[i, dsj] = (vs[j] - mean) * rstd * g_v[dsj] + b_v[dsj]

            pltpu.sync_copy(rows_v, emb_out.at[pl.ds(base, P)])
            return 0

        lax.fori_loop(0, B, batch_body, 0)

    return k


def kernel(input_ids, token_type_ids, word_emb, pos_emb, type_emb, gamma, beta):
    B, S = input_ids.shape
    V, H = word_emb.shape
    P = S // (NC * NS)
    ids_flat = input_ids.reshape(-1).astype(jnp.int32)
    tt_flat = token_type_ids.reshape(-1).astype(jnp.float32)
    k = _make_sc_kernel(B, S, V, H, P)
    emb, words = k(ids_flat, tt_flat, word_emb, pos_emb, type_emb, gamma, beta)
    return emb.reshape(B, S, H), words.reshape(B, S, H)
